# Initial kernel scaffold; baseline (speedup 1.0000x reference)
import jax
import jax.numpy as jnp
from jax.experimental import pallas as pl


def kernel(inputs, sep_token_indices, seg_emb1, seg_emb2):
    raise NotImplementedError("write your pallas kernel here")


# speedup vs baseline: 7.8251x; 7.8251x over previous
import functools

import jax
import jax.numpy as jnp
from jax import lax
from jax.experimental import pallas as pl
from jax.experimental.pallas import tpu as pltpu
from jax.experimental.pallas import tpu_sc as plsc

B, L, V, D = 4096, 200, 8192, 64
BL = B * L
NC, NS = 2, 16
NW = NC * NS
ROWS_W = B // NW
POS_W = ROWS_W * L
CHUNK = 128
NCHUNK = POS_W // CHUNK
VPC = CHUNK // 16


@functools.partial(
    pl.kernel,
    mesh=plsc.VectorSubcoreMesh(core_axis_name="c", subcore_axis_name="s"),
    out_type=jax.ShapeDtypeStruct((BL, D), jnp.float32),
    scratch_types=[
        pltpu.VMEM((NCHUNK, CHUNK), jnp.int32),
        pltpu.VMEM((ROWS_W,), jnp.int32),
        pltpu.VMEM((CHUNK, D), jnp.float32),
        pltpu.SemaphoreType.DMA,
    ],
    compiler_params=pltpu.CompilerParams(
        needs_layout_passes=False, use_tc_tiling_on_sc=False
    ),
)
def _seg_embed(table_hbm, tok_hbm, sep_hbm, out_hbm, idx_v, sep_v, rows_v, sem):
    wid = lax.axis_index("s") * NC + lax.axis_index("c")
    pltpu.sync_copy(tok_hbm.at[pl.ds(wid * NCHUNK, NCHUNK)], idx_v)
    pltpu.sync_copy(sep_hbm.at[pl.ds(wid * ROWS_W, ROWS_W)], sep_v)

    lanes = lax.iota(jnp.int32, 16)

    def chunk_body(j, carry):
        for k in range(VPC):
            col = k * 16
            p = j * CHUNK + col + lanes
            l_pos = lax.rem(p, L)
            row = lax.div(p, L)
            sep16 = plsc.load_gather(sep_v, [row])
            tok = idx_v[j, pl.ds(col, 16)]
            idx_v[j, pl.ds(col, 16)] = jnp.where(l_pos > sep16, tok + V, tok)
        pltpu.async_copy(table_hbm.at[idx_v.at[j]], rows_v, sem).wait()
        base = pl.multiple_of(wid * POS_W + j * CHUNK, CHUNK)
        pltpu.sync_copy(rows_v, out_hbm.at[pl.ds(base, CHUNK)])
        return carry

    lax.fori_loop(0, NCHUNK, chunk_body, 0)


def kernel(inputs, sep_token_indices, seg_emb1, seg_emb2):
    table = jnp.concatenate([seg_emb1, seg_emb2], axis=0)
    tok = inputs.astype(jnp.int32).reshape(BL // CHUNK, CHUNK)
    sep = sep_token_indices.astype(jnp.int32)
    out = _seg_embed(table, tok, sep)
    return out.reshape(B, L, D)


# speedup vs baseline: 9.4368x; 1.2060x over previous
import functools

import jax
import jax.numpy as jnp
from jax import lax
from jax.experimental import pallas as pl
from jax.experimental.pallas import tpu as pltpu
from jax.experimental.pallas import tpu_sc as plsc

B, L, V, D = 4096, 200, 8192, 64
BL = B * L
NC, NS = 2, 16
NW = NC * NS
ROWS_W = B // NW
POS_W = ROWS_W * L
CHUNK = 128
NCHUNK = POS_W // CHUNK
VPC = CHUNK // 16


NB = 4
NOUTER = NCHUNK // NB


@functools.partial(
    pl.kernel,
    mesh=plsc.VectorSubcoreMesh(core_axis_name="c", subcore_axis_name="s"),
    out_type=jax.ShapeDtypeStruct((BL, D), jnp.float32),
    scratch_types=[
        pltpu.VMEM((NCHUNK, CHUNK), jnp.int32),
        pltpu.VMEM((ROWS_W,), jnp.int32),
        pltpu.VMEM((NB, CHUNK, D), jnp.float32),
        pltpu.SemaphoreType.DMA((NB,)),
        pltpu.SemaphoreType.DMA((NB,)),
    ],
    compiler_params=pltpu.CompilerParams(
        needs_layout_passes=False, use_tc_tiling_on_sc=False
    ),
)
def _seg_embed(table_hbm, tok_hbm, sep_hbm, out_hbm, idx_v, sep_v, rows_v,
               gsem, wsem):
    wid = lax.axis_index("s") * NC + lax.axis_index("c")
    pltpu.sync_copy(tok_hbm.at[pl.ds(wid * NCHUNK, NCHUNK)], idx_v)
    pltpu.sync_copy(sep_hbm.at[pl.ds(wid * ROWS_W, ROWS_W)], sep_v)

    lanes = lax.iota(jnp.int32, 16)

    def out_slice(j):
        base = pl.multiple_of(wid * POS_W + j * CHUNK, CHUNK)
        return out_hbm.at[pl.ds(base, CHUNK)]

    def outer(jo, carry):
        for b in range(NB):
            j = jo * NB + b
            for k in range(VPC):
                col = k * 16
                p = j * CHUNK + col + lanes
                l_pos = lax.rem(p, L)
                row = lax.div(p, L)
                sep16 = plsc.load_gather(sep_v, [row])
                tok = idx_v[j, pl.ds(col, 16)]
                idx_v[j, pl.ds(col, 16)] = jnp.where(l_pos > sep16, tok + V, tok)
            @pl.when(jo > 0)
            def _():
                pltpu.make_async_copy(
                    rows_v.at[b], out_slice(j - NB), wsem.at[b]).wait()
            pltpu.make_async_copy(
                table_hbm.at[idx_v.at[j]], rows_v.at[b], gsem.at[b]).start()
        for b in range(NB):
            j = jo * NB + b
            pltpu.make_async_copy(
                table_hbm.at[idx_v.at[j]], rows_v.at[b], gsem.at[b]).wait()
            pltpu.make_async_copy(rows_v.at[b], out_slice(j), wsem.at[b]).start()
        return carry

    lax.fori_loop(0, NOUTER, outer, 0)
    for b in range(NB):
        j = (NOUTER - 1) * NB + b
        pltpu.make_async_copy(rows_v.at[b], out_slice(j), wsem.at[b]).wait()


def kernel(inputs, sep_token_indices, seg_emb1, seg_emb2):
    table = jnp.concatenate([seg_emb1, seg_emb2], axis=0)
    tok = inputs.astype(jnp.int32).reshape(BL // CHUNK, CHUNK)
    sep = sep_token_indices.astype(jnp.int32)
    out = _seg_embed(table, tok, sep)
    return out.reshape(B, L, D)
